# baseline (speedup 1.0000x reference)
import functools

import jax
import jax.numpy as jnp
from jax import lax
from jax.experimental import pallas as pl
from jax.experimental.pallas import tpu as pltpu
from jax.experimental.pallas import tpu_sc as plsc

OUT_D = 4096
ROWS = 5
B = 1024
NW = 32
PER_W = B // NW
GROUPS = OUT_D // 16
UNROLL = 8


def _compute_batch(j, floats_v, colf, buf):
    v = floats_v[pl.ds(2 * j, 16)]
    beg = jnp.full((16,), v[0], jnp.float32)
    end = jnp.full((16,), v[1], jnp.float32)
    step = (end - beg) * (1.0 / 4095.0)
    lin0 = colf[pl.ds(0, 16)] * step + beg
    bump = step * 16.0

    def col_group(g, lin):
        base = g * (16 * UNROLL)
        for u in range(UNROLL):
            off = base + u * 16
            buf[0, pl.ds(off, 16)] = jnp.maximum(0.0, jnp.minimum(lin, 2.0 - lin))
            buf[1, pl.ds(off, 16)] = jnp.maximum(0.0, lin - 1.0)
            buf[4, pl.ds(off, 16)] = jnp.maximum(0.0, -lin)
            lin = lin + bump
        return lin

    lax.fori_loop(0, GROUPS // UNROLL, col_group, lin0)


def _sc_body(floats_hbm, out_hbm, floats_v, colf, buf0, buf1, sem0, sem1):
    wid = lax.axis_index("s") * 2 + lax.axis_index("c")
    base = wid * PER_W
    pltpu.sync_copy(floats_hbm.at[pl.ds(base * 2, PER_W * 2)],
                    floats_v.at[pl.ds(0, PER_W * 2)])

    def init_group(g, _):
        gbase = g * (16 * UNROLL)
        for u in range(UNROLL):
            off = gbase + u * 16
            i16 = lax.broadcasted_iota(jnp.int32, (16,), 0) + off
            colf[pl.ds(off, 16)] = i16.astype(jnp.float32)
            z = jnp.zeros((16,), jnp.float32)
            buf0[2, pl.ds(off, 16)] = z
            buf0[3, pl.ds(off, 16)] = z
            buf1[2, pl.ds(off, 16)] = z
            buf1[3, pl.ds(off, 16)] = z
        return 0

    lax.fori_loop(0, GROUPS // UNROLL, init_group, 0)

    def outer(jj, _):
        @pl.when(jj > 0)
        def _w0():
            pltpu.make_async_copy(buf0, out_hbm.at[base], sem0).wait()

        _compute_batch(2 * jj, floats_v, colf, buf0)
        pltpu.async_copy(buf0, out_hbm.at[base + 2 * jj], sem0)

        @pl.when(jj > 0)
        def _w1():
            pltpu.make_async_copy(buf1, out_hbm.at[base], sem1).wait()

        _compute_batch(2 * jj + 1, floats_v, colf, buf1)
        pltpu.async_copy(buf1, out_hbm.at[base + 2 * jj + 1], sem1)
        return 0

    lax.fori_loop(0, PER_W // 2, outer, 0)
    pltpu.make_async_copy(buf0, out_hbm.at[base], sem0).wait()
    pltpu.make_async_copy(buf1, out_hbm.at[base], sem1).wait()


_sc_kernel = functools.partial(
    pl.kernel,
    out_type=jax.ShapeDtypeStruct((B, ROWS, OUT_D), jnp.float32),
    mesh=plsc.VectorSubcoreMesh(core_axis_name="c", subcore_axis_name="s"),
    scratch_types=[
        pltpu.VMEM((PER_W * 2 + 32, ), jnp.float32),
        pltpu.VMEM((OUT_D,), jnp.float32),
        pltpu.VMEM((ROWS, OUT_D), jnp.float32),
        pltpu.VMEM((ROWS, OUT_D), jnp.float32),
        pltpu.SemaphoreType.DMA,
        pltpu.SemaphoreType.DMA,
    ],
)(_sc_body)


def kernel(floats):
    bsz = floats.shape[0]
    mats = _sc_kernel(floats.reshape(-1))
    return (mats, jnp.ones((bsz, 1), jnp.float32))


# speedup vs baseline: 1.0023x; 1.0023x over previous
import functools

import jax
import jax.numpy as jnp
from jax import lax
from jax.experimental import pallas as pl
from jax.experimental.pallas import tpu as pltpu
from jax.experimental.pallas import tpu_sc as plsc

OUT_D = 4096
ROWS = 5
B = 1024
NW = 32
PER_W = B // NW
GROUPS = OUT_D // 16
UNROLL = 8


def _compute_batch(j, floats_v, buf):
    v = floats_v[pl.ds(2 * j, 16)]
    beg = jnp.full((16,), v[0], jnp.float32)
    end = jnp.full((16,), v[1], jnp.float32)
    step = (end - beg) * (1.0 / 4095.0)
    i16 = lax.broadcasted_iota(jnp.int32, (16,), 0).astype(jnp.float32)
    lin0 = i16 * step + beg
    bump = step * 16.0

    def col_group(g, lin):
        base = g * (16 * UNROLL)
        for u in range(UNROLL):
            off = base + u * 16
            buf[0, pl.ds(off, 16)] = jnp.maximum(0.0, jnp.minimum(lin, 2.0 - lin))
            buf[1, pl.ds(off, 16)] = jnp.maximum(0.0, lin - 1.0)
            buf[4, pl.ds(off, 16)] = jnp.maximum(0.0, -lin)
            lin = lin + bump
        return lin

    lax.fori_loop(0, GROUPS // UNROLL, col_group, lin0)


def _sc_body(floats_hbm, out_hbm, floats_v, buf0, buf1, sem0, sem1):
    wid = lax.axis_index("s") * 2 + lax.axis_index("c")
    base = wid * PER_W
    pltpu.sync_copy(floats_hbm.at[pl.ds(base * 2, PER_W * 2)],
                    floats_v.at[pl.ds(0, PER_W * 2)])

    def init_group(g, _):
        gbase = g * (16 * UNROLL)
        for u in range(UNROLL):
            off = gbase + u * 16
            z = jnp.zeros((16,), jnp.float32)
            buf0[2, pl.ds(off, 16)] = z
            buf0[3, pl.ds(off, 16)] = z
            buf1[2, pl.ds(off, 16)] = z
            buf1[3, pl.ds(off, 16)] = z
        return 0

    lax.fori_loop(0, GROUPS // UNROLL, init_group, 0)

    def outer(jj, _):
        @pl.when(jj > 0)
        def _w0():
            pltpu.make_async_copy(buf0, out_hbm.at[base], sem0).wait()

        _compute_batch(2 * jj, floats_v, buf0)
        pltpu.async_copy(buf0, out_hbm.at[base + 2 * jj], sem0)

        @pl.when(jj > 0)
        def _w1():
            pltpu.make_async_copy(buf1, out_hbm.at[base], sem1).wait()

        _compute_batch(2 * jj + 1, floats_v, buf1)
        pltpu.async_copy(buf1, out_hbm.at[base + 2 * jj + 1], sem1)
        return 0

    lax.fori_loop(0, PER_W // 2, outer, 0)
    pltpu.make_async_copy(buf0, out_hbm.at[base], sem0).wait()
    pltpu.make_async_copy(buf1, out_hbm.at[base], sem1).wait()


_sc_kernel = functools.partial(
    pl.kernel,
    out_type=jax.ShapeDtypeStruct((B, ROWS, OUT_D), jnp.float32),
    mesh=plsc.VectorSubcoreMesh(core_axis_name="c", subcore_axis_name="s"),
    scratch_types=[
        pltpu.VMEM((PER_W * 2 + 32, ), jnp.float32),
        pltpu.VMEM((ROWS, OUT_D), jnp.float32),
        pltpu.VMEM((ROWS, OUT_D), jnp.float32),
        pltpu.SemaphoreType.DMA,
        pltpu.SemaphoreType.DMA,
    ],
)(_sc_body)


def kernel(floats):
    bsz = floats.shape[0]
    mats = _sc_kernel(floats.reshape(-1))
    return (mats, jnp.ones((bsz, 1), jnp.float32))
